# Initial kernel scaffold; baseline (speedup 1.0000x reference)
import jax
import jax.numpy as jnp
from jax.experimental import pallas as pl


def kernel(x, edge_index, Wp, bp, W1l, W1r, b1, W2l, W2r, b2, W3l, W3r, b3):
    raise NotImplementedError("write your pallas kernel here")


# speedup vs baseline: 2.8002x; 2.8002x over previous
import functools

import jax
import jax.numpy as jnp
from jax import lax
from jax.experimental import pallas as pl
from jax.experimental.pallas import tpu as pltpu
from jax.experimental.pallas import tpu_sc as plsc

NC = 2
NS = 16
NW = NC * NS
L = 16
CH = 128
IB = 4
BR = 1024


def _sc_mesh():
    return plsc.VectorSubcoreMesh(
        core_axis_name="c", subcore_axis_name="s", num_cores=NC, num_subcores=NS
    )


def _make_sc_agg(n_pad, h, k):
    out_type = jax.ShapeDtypeStruct((NC, n_pad, h), jnp.float32)
    scratch = [
        pltpu.VMEM((IB, CH), jnp.int32),
        pltpu.VMEM((IB, CH), jnp.int32),
        pltpu.VMEM((CH, h), jnp.float32),
        pltpu.SemaphoreType.DMA,
        pltpu.VMEM_SHARED((n_pad, h), jnp.float32),
    ]

    def body(y_hbm, srcs_hbm, dsts_hbm, zeros_hbm, out_hbm,
             src_v, dst_v, rows_v, sem, acc_sh):
        c = lax.axis_index("c")
        s = lax.axis_index("s")
        wid = c * NS + s

        rpt = n_pad // NS
        pltpu.sync_copy(
            zeros_hbm.at[pl.ds(s * rpt, rpt)], acc_sh.at[pl.ds(s * rpt, rpt)]
        )

        plsc.subcore_barrier()

        def gbody(g, carry):
            pltpu.sync_copy(srcs_hbm.at[wid, pl.ds(g * IB, IB)], src_v)
            pltpu.sync_copy(dsts_hbm.at[wid, pl.ds(g * IB, IB)], dst_v)
            for j in range(IB):
                pltpu.async_copy(y_hbm.at[src_v.at[j]], rows_v, sem).wait()
                pltpu.sync_copy(rows_v, acc_sh.at[dst_v.at[j]], add=True)
            return carry

        lax.fori_loop(0, k // IB, gbody, 0)

        plsc.subcore_barrier()

        pltpu.sync_copy(
            acc_sh.at[pl.ds(s * rpt, rpt)], out_hbm.at[c, pl.ds(s * rpt, rpt)]
        )

    return pl.kernel(
        body,
        out_type=out_type,
        mesh=_sc_mesh(),
        scratch_types=scratch,
    )


def _dot(a, b):
    return jnp.dot(a, b, preferred_element_type=jnp.float32)


def _tc1_body(x_r, wp_r, bp_r, w1l_r, w1r_r, b1_r, xp_o, y1_o, z1_o):
    xb = x_r[...]
    xp_o[...] = jnp.maximum(_dot(xb, wp_r[...]) + bp_r[...], 0.0)
    y1_o[...] = _dot(xb, w1l_r[...])
    z1_o[...] = _dot(xb, w1r_r[...]) + b1_r[...]


def _inv_deg(cnt_r):
    deg = cnt_r[0, :, 0] + cnt_r[1, :, 0]
    return 1.0 / jnp.maximum(deg, 1.0)


def _tc2_body(agg_r, cnt_r, z1_r, xp_r, w2la_r, w2lb_r, w2ra_r, w2rb_r, b2_r,
              h1_o, y2_o, z2_o):
    inv = _inv_deg(cnt_r)
    mean = (agg_r[0] + agg_r[1]) * inv[:, None]
    h1 = jnp.maximum(mean + z1_r[...], 0.0)
    h1_o[...] = h1
    xpb = xp_r[...]
    y2_o[...] = _dot(xpb, w2la_r[...]) + _dot(h1, w2lb_r[...])
    z2_o[...] = _dot(xpb, w2ra_r[...]) + _dot(h1, w2rb_r[...]) + b2_r[...]


def _tc3_body(agg_r, cnt_r, z2_r, xp_r, h1_r,
              w3la_r, w3lb_r, w3lc_r, w3ra_r, w3rb_r, w3rc_r, b3_r,
              y3_o, z3_o):
    inv = _inv_deg(cnt_r)
    mean = (agg_r[0] + agg_r[1]) * inv[:, None]
    h2 = jnp.maximum(mean + z2_r[...], 0.0)
    xpb = xp_r[...]
    h1b = h1_r[...]
    y3_o[...] = _dot(xpb, w3la_r[...]) + _dot(h1b, w3lb_r[...]) + _dot(h2, w3lc_r[...])
    z3_o[...] = (_dot(xpb, w3ra_r[...]) + _dot(h1b, w3rb_r[...])
                 + _dot(h2, w3rc_r[...]) + b3_r[...])


def _tc4_body(agg_r, cnt_r, z3_r, h3_o):
    inv = _inv_deg(cnt_r)
    mean = (agg_r[0] + agg_r[1]) * inv[:, None]
    h3_o[...] = jnp.maximum(mean + z3_r[...], 0.0)


def _row_spec(h):
    return pl.BlockSpec((BR, h), lambda i: (i, 0))


def _full_spec(shape):
    nd = len(shape)
    return pl.BlockSpec(shape, lambda i: (0,) * nd)


def _agg_spec(h):
    return pl.BlockSpec((NC, BR, h), lambda i: (0, i, 0))


def _cnt_spec():
    return pl.BlockSpec((NC, BR, L), lambda i: (0, i, 0))


def kernel(x, edge_index, Wp, bp, W1l, W1r, b1, W2l, W2r, b2, W3l, W3r, b3):
    n, d = x.shape
    h = Wp.shape[1]
    e = edge_index.shape[1]

    k = -(-e // (NW * CH * IB)) * IB
    e_pad = NW * CH * k
    n_pad = -(-(n + 1) // (NS * 8)) * (NS * 8)

    src = jnp.concatenate(
        [edge_index[0], jnp.zeros((e_pad - e,), jnp.int32)]).reshape(NW, k, CH)
    dst = jnp.concatenate(
        [edge_index[1], jnp.full((e_pad - e,), n, jnp.int32)]).reshape(NW, k, CH)
    zeros_acc = jnp.zeros((n_pad, h), jnp.float32)
    ones_feat = jnp.ones((n, h), jnp.float32)

    grid = -(-n // BR)
    bp2, b12, b22, b32 = (v.reshape(1, h) for v in (bp, b1, b2, b3))

    xp, y1, z1 = pl.pallas_call(
        _tc1_body,
        grid=(grid,),
        in_specs=[_row_spec(d)] + [_full_spec(s.shape) for s in (Wp, bp2, W1l, W1r, b12)],
        out_specs=[_row_spec(h)] * 3,
        out_shape=[jax.ShapeDtypeStruct((n, h), jnp.float32)] * 3,
    )(x, Wp, bp2, W1l, W1r, b12)

    sc_agg = _make_sc_agg(n_pad, h, k)

    cnt = sc_agg(ones_feat, src, dst, zeros_acc)[:, :, :L]
    agg1 = sc_agg(y1, src, dst, zeros_acc)

    h1, y2, z2 = pl.pallas_call(
        _tc2_body,
        grid=(grid,),
        in_specs=[_agg_spec(h), _cnt_spec(), _row_spec(h), _row_spec(h)]
        + [_full_spec((h, h))] * 4 + [_full_spec((1, h))],
        out_specs=[_row_spec(h)] * 3,
        out_shape=[jax.ShapeDtypeStruct((n, h), jnp.float32)] * 3,
    )(agg1, cnt, z1, xp, W2l[:h], W2l[h:], W2r[:h], W2r[h:], b22)

    agg2 = sc_agg(y2, src, dst, zeros_acc)

    y3, z3 = pl.pallas_call(
        _tc3_body,
        grid=(grid,),
        in_specs=[_agg_spec(h), _cnt_spec()] + [_row_spec(h)] * 3
        + [_full_spec((h, h))] * 6 + [_full_spec((1, h))],
        out_specs=[_row_spec(h)] * 2,
        out_shape=[jax.ShapeDtypeStruct((n, h), jnp.float32)] * 2,
    )(agg2, cnt, z2, xp, h1,
      W3l[:h], W3l[h:2 * h], W3l[2 * h:],
      W3r[:h], W3r[h:2 * h], W3r[2 * h:], b32)

    agg3 = sc_agg(y3, src, dst, zeros_acc)

    h3 = pl.pallas_call(
        _tc4_body,
        grid=(grid,),
        in_specs=[_agg_spec(h), _cnt_spec(), _row_spec(h)],
        out_specs=_row_spec(h),
        out_shape=jax.ShapeDtypeStruct((n, h), jnp.float32),
    )(agg3, cnt, z3)

    return h3


# speedup vs baseline: 2.9924x; 1.0687x over previous
import functools

import jax
import jax.numpy as jnp
from jax import lax
from jax.experimental import pallas as pl
from jax.experimental.pallas import tpu as pltpu
from jax.experimental.pallas import tpu_sc as plsc

NC = 2
NS = 16
NW = NC * NS
L = 16
CH = 128
IB = 4
BR = 1024


def _sc_mesh():
    return plsc.VectorSubcoreMesh(
        core_axis_name="c", subcore_axis_name="s", num_cores=NC, num_subcores=NS
    )


def _make_sc_agg(n_pad, h, k):
    out_type = jax.ShapeDtypeStruct((NC, n_pad, h), jnp.float32)
    scratch = [
        pltpu.VMEM((IB, CH), jnp.int32),
        pltpu.VMEM((IB, CH), jnp.int32),
        pltpu.VMEM((2, CH, h), jnp.float32),
        [pltpu.SemaphoreType.DMA] * 2,
        [pltpu.SemaphoreType.DMA] * 2,
        pltpu.VMEM_SHARED((n_pad, h), jnp.float32),
    ]

    def body(y_hbm, srcs_hbm, dsts_hbm, zeros_hbm, out_hbm,
             src_v, dst_v, rows_v, sem_g, sem_s, acc_sh):
        c = lax.axis_index("c")
        s = lax.axis_index("s")
        wid = c * NS + s

        rpt = n_pad // NS
        pltpu.sync_copy(
            zeros_hbm.at[pl.ds(s * rpt, rpt)], acc_sh.at[pl.ds(s * rpt, rpt)]
        )

        plsc.subcore_barrier()

        def gbody(g, carry):
            pltpu.sync_copy(srcs_hbm.at[wid, pl.ds(g * IB, IB)], src_v)
            pltpu.sync_copy(dsts_hbm.at[wid, pl.ds(g * IB, IB)], dst_v)
            dg = [None, None]
            ds = [None, None]
            for j in range(IB):
                b = j % 2
                if ds[b] is not None:
                    ds[b].wait()
                dg[b] = pltpu.async_copy(
                    y_hbm.at[src_v.at[j]], rows_v.at[b], sem_g[b]
                )
                if j >= 1:
                    p = (j - 1) % 2
                    dg[p].wait()
                    ds[p] = pltpu.async_copy(
                        rows_v.at[p], acc_sh.at[dst_v.at[j - 1]], sem_s[p],
                        add=True,
                    )
            bl = (IB - 1) % 2
            dg[bl].wait()
            ds[bl] = pltpu.async_copy(
                rows_v.at[bl], acc_sh.at[dst_v.at[IB - 1]], sem_s[bl], add=True
            )
            ds[1 - bl].wait()
            ds[bl].wait()
            return carry

        lax.fori_loop(0, k // IB, gbody, 0)

        plsc.subcore_barrier()

        pltpu.sync_copy(
            acc_sh.at[pl.ds(s * rpt, rpt)], out_hbm.at[c, pl.ds(s * rpt, rpt)]
        )

    return pl.kernel(
        body,
        out_type=out_type,
        mesh=_sc_mesh(),
        scratch_types=scratch,
    )


def _dot(a, b):
    return jnp.dot(a, b, preferred_element_type=jnp.float32)


def _tc1_body(x_r, wp_r, bp_r, w1l_r, w1r_r, b1_r, xp_o, y1_o, z1_o):
    xb = x_r[...]
    xp_o[...] = jnp.maximum(_dot(xb, wp_r[...]) + bp_r[...], 0.0)
    y1_o[...] = _dot(xb, w1l_r[...])
    z1_o[...] = _dot(xb, w1r_r[...]) + b1_r[...]


def _inv_deg(cnt_r):
    deg = cnt_r[0, :, 0] + cnt_r[1, :, 0]
    return 1.0 / jnp.maximum(deg, 1.0)


def _tc2_body(agg_r, cnt_r, z1_r, xp_r, w2la_r, w2lb_r, w2ra_r, w2rb_r, b2_r,
              h1_o, y2_o, z2_o):
    inv = _inv_deg(cnt_r)
    mean = (agg_r[0] + agg_r[1]) * inv[:, None]
    h1 = jnp.maximum(mean + z1_r[...], 0.0)
    h1_o[...] = h1
    xpb = xp_r[...]
    y2_o[...] = _dot(xpb, w2la_r[...]) + _dot(h1, w2lb_r[...])
    z2_o[...] = _dot(xpb, w2ra_r[...]) + _dot(h1, w2rb_r[...]) + b2_r[...]


def _tc3_body(agg_r, cnt_r, z2_r, xp_r, h1_r,
              w3la_r, w3lb_r, w3lc_r, w3ra_r, w3rb_r, w3rc_r, b3_r,
              y3_o, z3_o):
    inv = _inv_deg(cnt_r)
    mean = (agg_r[0] + agg_r[1]) * inv[:, None]
    h2 = jnp.maximum(mean + z2_r[...], 0.0)
    xpb = xp_r[...]
    h1b = h1_r[...]
    y3_o[...] = _dot(xpb, w3la_r[...]) + _dot(h1b, w3lb_r[...]) + _dot(h2, w3lc_r[...])
    z3_o[...] = (_dot(xpb, w3ra_r[...]) + _dot(h1b, w3rb_r[...])
                 + _dot(h2, w3rc_r[...]) + b3_r[...])


def _tc4_body(agg_r, cnt_r, z3_r, h3_o):
    inv = _inv_deg(cnt_r)
    mean = (agg_r[0] + agg_r[1]) * inv[:, None]
    h3_o[...] = jnp.maximum(mean + z3_r[...], 0.0)


def _row_spec(h):
    return pl.BlockSpec((BR, h), lambda i: (i, 0))


def _full_spec(shape):
    nd = len(shape)
    return pl.BlockSpec(shape, lambda i: (0,) * nd)


def _agg_spec(h):
    return pl.BlockSpec((NC, BR, h), lambda i: (0, i, 0))


def _cnt_spec():
    return pl.BlockSpec((NC, BR, L), lambda i: (0, i, 0))


def kernel(x, edge_index, Wp, bp, W1l, W1r, b1, W2l, W2r, b2, W3l, W3r, b3):
    n, d = x.shape
    h = Wp.shape[1]
    e = edge_index.shape[1]

    k = -(-e // (NW * CH * IB)) * IB
    e_pad = NW * CH * k
    n_pad = -(-(n + 1) // (NS * 8)) * (NS * 8)

    src = jnp.concatenate(
        [edge_index[0], jnp.zeros((e_pad - e,), jnp.int32)]).reshape(NW, k, CH)
    dst = jnp.concatenate(
        [edge_index[1], jnp.full((e_pad - e,), n, jnp.int32)]).reshape(NW, k, CH)
    zeros_acc = jnp.zeros((n_pad, h), jnp.float32)
    ones_feat = jnp.ones((n, h), jnp.float32)

    grid = -(-n // BR)
    bp2, b12, b22, b32 = (v.reshape(1, h) for v in (bp, b1, b2, b3))

    xp, y1, z1 = pl.pallas_call(
        _tc1_body,
        grid=(grid,),
        in_specs=[_row_spec(d)] + [_full_spec(s.shape) for s in (Wp, bp2, W1l, W1r, b12)],
        out_specs=[_row_spec(h)] * 3,
        out_shape=[jax.ShapeDtypeStruct((n, h), jnp.float32)] * 3,
    )(x, Wp, bp2, W1l, W1r, b12)

    sc_agg = _make_sc_agg(n_pad, h, k)

    cnt = sc_agg(ones_feat, src, dst, zeros_acc)[:, :, :L]
    agg1 = sc_agg(y1, src, dst, zeros_acc)

    h1, y2, z2 = pl.pallas_call(
        _tc2_body,
        grid=(grid,),
        in_specs=[_agg_spec(h), _cnt_spec(), _row_spec(h), _row_spec(h)]
        + [_full_spec((h, h))] * 4 + [_full_spec((1, h))],
        out_specs=[_row_spec(h)] * 3,
        out_shape=[jax.ShapeDtypeStruct((n, h), jnp.float32)] * 3,
    )(agg1, cnt, z1, xp, W2l[:h], W2l[h:], W2r[:h], W2r[h:], b22)

    agg2 = sc_agg(y2, src, dst, zeros_acc)

    y3, z3 = pl.pallas_call(
        _tc3_body,
        grid=(grid,),
        in_specs=[_agg_spec(h), _cnt_spec()] + [_row_spec(h)] * 3
        + [_full_spec((h, h))] * 6 + [_full_spec((1, h))],
        out_specs=[_row_spec(h)] * 2,
        out_shape=[jax.ShapeDtypeStruct((n, h), jnp.float32)] * 2,
    )(agg2, cnt, z2, xp, h1,
      W3l[:h], W3l[h:2 * h], W3l[2 * h:],
      W3r[:h], W3r[h:2 * h], W3r[2 * h:], b32)

    agg3 = sc_agg(y3, src, dst, zeros_acc)

    h3 = pl.pallas_call(
        _tc4_body,
        grid=(grid,),
        in_specs=[_agg_spec(h), _cnt_spec(), _row_spec(h)],
        out_specs=_row_spec(h),
        out_shape=jax.ShapeDtypeStruct((n, h), jnp.float32),
    )(agg3, cnt, z3)

    return h3
